# baseline (speedup 1.0000x reference)
import functools

import jax
import jax.numpy as jnp
from jax import lax
from jax.experimental import pallas as pl
from jax.experimental.pallas import tpu as pltpu
from jax.experimental.pallas import tpu_sc as plsc

_NC, _NS = 2, 16
_NW = _NC * _NS
_CHUNK = 128
_L = 16


def _tc_dist_body(x_ref, w_ref, idxm_ref, idxg_ref, sc_ref,
                  wt2_ref, w2_ref, acc_ref, *, nb, k, bm):
    i = pl.program_id(0)

    @pl.when(i == 0)
    def _init():
        acc_ref[0] = 0.0
        wt2_ref[...] = jnp.transpose(w_ref[...]) * 2.0
        wt0 = wt2_ref[...]
        w2_ref[...] = jnp.sum(wt0 * wt0, axis=0, keepdims=True) * 0.25

    x = x_ref[...]
    x2 = jnp.sum(x * x, axis=1, keepdims=True)
    xw2 = lax.dot_general(x, wt2_ref[...], (((1,), (0,)), ((), ())),
                          preferred_element_type=jnp.float32)
    dist = (x2 + w2_ref[...]) - xw2

    idx = jnp.argmin(dist, axis=1, keepdims=True)
    valid = jnp.sqrt(x2) > 1e-06

    idx_t = jnp.reshape(idx, (bm,))
    valid_t = jnp.reshape(valid, (bm,))

    idxm_ref[pl.ds(i * bm, bm)] = jnp.where(valid_t, idx_t, 0)
    idxg_ref[pl.ds(i * bm, bm)] = jnp.where(valid_t, idx_t, k)
    acc_ref[0] += jnp.sum(valid_t.astype(jnp.float32))

    @pl.when(i == nb - 1)
    def _fin():
        sc_ref[0] = jnp.maximum(acc_ref[0], 1.0)


def _sc_gather_body(idxg_hbm, w_hbm, x_hbm, qst_hbm, hist_hbm, md_hbm, sq_hbm,
                    idx_v, idx1, rows_v, x_v, md_v, sq_v, ones_v, zeros_v,
                    hist_sh, sem, *, rpw, hist_pad, k, d):
    cid = lax.axis_index("c")
    sid = lax.axis_index("s")
    wid = sid * _NC + cid
    base = wid * rpw
    nchunks = rpw // _CHUNK
    ngroups = rpw // _L
    stripe = hist_pad // _NS

    zero16 = jnp.zeros((_L,), jnp.float32)

    def _zero(i, carry):
        zeros_v[pl.ds(i * _L, _L)] = zero16
        return carry

    lax.fori_loop(0, stripe // _L, _zero, 0)
    pltpu.sync_copy(zeros_v, hist_sh.at[pl.ds(sid * stripe, stripe)])

    one16 = jnp.ones((_L,), jnp.float32)

    def _ones(i, carry):
        ones_v[pl.ds(i * _L, _L)] = one16
        return carry

    lax.fori_loop(0, _CHUNK // _L, _ones, 0)

    for c in range(nchunks):
        pltpu.sync_copy(idxg_hbm.at[pl.ds(base + c * _CHUNK, _CHUNK)],
                        idx_v.at[c])
    pltpu.sync_copy(idxg_hbm.at[pl.ds(base, rpw)], idx1)
    pltpu.sync_copy(x_hbm.at[pl.ds(base * d, rpw * d)], x_v)
    for c in range(nchunks):
        pltpu.async_copy(w_hbm.at[idx_v.at[c]],
                         rows_v.at[pl.ds(c * _CHUNK, _CHUNK)], sem).wait()
    pltpu.sync_copy(rows_v, qst_hbm.at[pl.ds(base, rpw)])

    iota16 = lax.broadcasted_iota(jnp.int32, (_L,), 0)

    def _group(g, sqacc):
        row16 = iota16 + g * _L
        xoff = row16 * d

        md16 = jnp.zeros((_L,), jnp.float32)
        for j in range(d):
            colj = jnp.zeros((_L,), jnp.int32) + j
            xj = plsc.load_gather(x_v, [xoff + colj])
            qj = plsc.load_gather(rows_v, [row16, colj])
            dj = xj - qj
            md16 = md16 + dj * dj
        idx16 = idx1[pl.ds(g * _L, _L)]
        md16 = jnp.where(idx16 < k, md16, 0.0)
        md_v[pl.ds(g * _L, _L)] = md16
        return sqacc + md16

    sqacc = lax.fori_loop(0, ngroups, _group, jnp.zeros((_L,), jnp.float32))
    sq_v[...] = sqacc
    pltpu.sync_copy(sq_v, sq_hbm.at[wid])
    pltpu.sync_copy(md_v, md_hbm.at[pl.ds(base, rpw)])

    plsc.subcore_barrier()
    for c in range(nchunks):
        pltpu.sync_copy(ones_v, hist_sh.at[idx_v.at[c]], add=True)
    plsc.subcore_barrier()

    @pl.when(sid == 0)
    def _emit():
        pltpu.sync_copy(hist_sh, hist_hbm.at[cid])


def _tc_final_body(hist_ref, sq_ref, sc1_ref, out_ref, *, k, d):
    counts = jnp.sum(hist_ref[...][:, :k], axis=0, keepdims=True)
    nv = sc1_ref[0]
    loss_vq = jnp.sum(sq_ref[...]) / (nv * d)
    p = counts / nv
    entropy = jnp.sum(p * jnp.log(p + 1e-10))
    perplexity = jnp.exp(-entropy)
    perplexity_loss = -jnp.log(perplexity + 1e-10)
    out_ref[0] = loss_vq + 0.01 * perplexity_loss
    out_ref[1] = loss_vq
    out_ref[2] = perplexity_loss
    out_ref[3] = perplexity


@jax.jit
def kernel(inputs, W):
    d = W.shape[1]
    K = W.shape[0]
    flat = inputs.reshape(-1, d)
    M = flat.shape[0]
    BM = 256
    nb = M // BM
    rpw = M // _NW
    hist_pad = K + 512

    idxm, idxg, sc1 = pl.pallas_call(
        functools.partial(_tc_dist_body, nb=nb, k=K, bm=BM),
        grid=(nb,),
        in_specs=[
            pl.BlockSpec((BM, d), lambda i: (i, 0)),
            pl.BlockSpec((K, d), lambda i: (0, 0)),
        ],
        out_specs=[
            pl.BlockSpec((M,), lambda i: (0,)),
            pl.BlockSpec((M,), lambda i: (0,)),
            pl.BlockSpec(memory_space=pltpu.SMEM),
        ],
        out_shape=[
            jax.ShapeDtypeStruct((M,), jnp.int32),
            jax.ShapeDtypeStruct((M,), jnp.int32),
            jax.ShapeDtypeStruct((1,), jnp.float32),
        ],
        scratch_shapes=[pltpu.VMEM((d, K), jnp.float32),
                        pltpu.VMEM((1, K), jnp.float32),
                        pltpu.SMEM((1,), jnp.float32)],
    )(flat, W)

    w_aug = jnp.concatenate([W, jnp.zeros((8, d), jnp.float32)], axis=0)
    x1d = inputs.reshape(-1)

    qst, hist, md, sq = pl.kernel(
        functools.partial(_sc_gather_body, rpw=rpw, hist_pad=hist_pad,
                          k=K, d=d),
        out_type=[
            jax.ShapeDtypeStruct((M, d), jnp.float32),
            jax.ShapeDtypeStruct((_NC, hist_pad), jnp.float32),
            jax.ShapeDtypeStruct((M,), jnp.float32),
            jax.ShapeDtypeStruct((_NW, _L), jnp.float32),
        ],
        mesh=plsc.VectorSubcoreMesh(core_axis_name="c", subcore_axis_name="s"),
        scratch_types=[
            pltpu.VMEM((rpw // _CHUNK, _CHUNK), jnp.int32),
            pltpu.VMEM((rpw,), jnp.int32),
            pltpu.VMEM((rpw, d), jnp.float32),
            pltpu.VMEM((rpw * d,), jnp.float32),
            pltpu.VMEM((rpw,), jnp.float32),
            pltpu.VMEM((_L,), jnp.float32),
            pltpu.VMEM((_CHUNK,), jnp.float32),
            pltpu.VMEM((hist_pad // _NS,), jnp.float32),
            pltpu.VMEM_SHARED((hist_pad,), jnp.float32),
            pltpu.SemaphoreType.DMA,
        ],
        compiler_params=pltpu.CompilerParams(use_tc_tiling_on_sc=False,
                                             needs_layout_passes=False),
    )(idxg, w_aug, x1d)

    sc3 = pl.pallas_call(
        functools.partial(_tc_final_body, k=K, d=d),
        in_specs=[
            pl.BlockSpec((_NC, hist_pad), lambda: (0, 0)),
            pl.BlockSpec((_NW, _L), lambda: (0, 0)),
            pl.BlockSpec(memory_space=pltpu.SMEM),
        ],
        out_specs=pl.BlockSpec(memory_space=pltpu.SMEM),
        out_shape=jax.ShapeDtypeStruct((4,), jnp.float32),
    )(hist, sq, sc1)

    return (qst.reshape(inputs.shape), sc3[0], idxm,
            idxm.reshape(inputs.shape[:-1]), md.reshape(inputs.shape[:-1]),
            sc3[1], sc3[2], sc3[3])


# speedup vs baseline: 1.0072x; 1.0072x over previous
import functools

import jax
import jax.numpy as jnp
from jax import lax
from jax.experimental import pallas as pl
from jax.experimental.pallas import tpu as pltpu
from jax.experimental.pallas import tpu_sc as plsc

_NC, _NS = 2, 16
_NW = _NC * _NS
_CHUNK = 128
_L = 16


def _tc_dist_body(x_ref, w_ref, idxm_ref, idxg_ref, sc_ref, waug_ref,
                  wt2_ref, w2_ref, acc_ref, *, nb, k, bm):
    i = pl.program_id(0)

    @pl.when(i == 0)
    def _init():
        acc_ref[0] = 0.0
        wt2_ref[...] = jnp.transpose(w_ref[...]) * 2.0
        wt0 = wt2_ref[...]
        w2_ref[...] = jnp.sum(wt0 * wt0, axis=0, keepdims=True) * 0.25
        waug_ref[...] = jnp.concatenate(
            [w_ref[...], jnp.zeros((8, w_ref.shape[1]), jnp.float32)], axis=0)

    x = x_ref[...]
    x2 = jnp.sum(x * x, axis=1, keepdims=True)
    xw2 = lax.dot_general(x, wt2_ref[...], (((1,), (0,)), ((), ())),
                          preferred_element_type=jnp.float32)
    dist = (x2 + w2_ref[...]) - xw2

    idx = jnp.argmin(dist, axis=1, keepdims=True)
    valid = jnp.sqrt(x2) > 1e-06

    idx_t = jnp.reshape(idx, (bm,))
    valid_t = jnp.reshape(valid, (bm,))

    idxm_ref[pl.ds(i * bm, bm)] = jnp.where(valid_t, idx_t, 0)
    idxg_ref[pl.ds(i * bm, bm)] = jnp.where(valid_t, idx_t, k)
    acc_ref[0] += jnp.sum(valid_t.astype(jnp.float32))

    @pl.when(i == nb - 1)
    def _fin():
        sc_ref[0] = jnp.maximum(acc_ref[0], 1.0)


def _sc_gather_body(idxg_hbm, w_hbm, x_hbm, qst_hbm, hist_hbm, md_hbm, sq_hbm,
                    idx_v, idx1, rows_v, x_v, md_v, sq_v, ones_v, zeros_v,
                    hist_sh, sem, *, rpw, hist_pad, k, d):
    cid = lax.axis_index("c")
    sid = lax.axis_index("s")
    wid = sid * _NC + cid
    base = wid * rpw
    nchunks = rpw // _CHUNK
    ngroups = rpw // _L
    stripe = hist_pad // _NS

    zero16 = jnp.zeros((_L,), jnp.float32)

    def _zero(i, carry):
        zeros_v[pl.ds(i * _L, _L)] = zero16
        return carry

    lax.fori_loop(0, stripe // _L, _zero, 0)
    pltpu.sync_copy(zeros_v, hist_sh.at[pl.ds(sid * stripe, stripe)])

    one16 = jnp.ones((_L,), jnp.float32)

    def _ones(i, carry):
        ones_v[pl.ds(i * _L, _L)] = one16
        return carry

    lax.fori_loop(0, _CHUNK // _L, _ones, 0)

    for c in range(nchunks):
        pltpu.sync_copy(idxg_hbm.at[pl.ds(base + c * _CHUNK, _CHUNK)],
                        idx_v.at[c])
    pltpu.sync_copy(idxg_hbm.at[pl.ds(base, rpw)], idx1)
    pltpu.sync_copy(x_hbm.at[pl.ds(base * d, rpw * d)], x_v)
    for c in range(nchunks):
        pltpu.async_copy(w_hbm.at[idx_v.at[c]],
                         rows_v.at[pl.ds(c * _CHUNK, _CHUNK)], sem).wait()
    pltpu.sync_copy(rows_v, qst_hbm.at[pl.ds(base, rpw)])

    iota16 = lax.broadcasted_iota(jnp.int32, (_L,), 0)

    def _group(g, sqacc):
        row16 = iota16 + g * _L
        xoff = row16 * d

        md16 = jnp.zeros((_L,), jnp.float32)
        for j in range(d):
            colj = jnp.zeros((_L,), jnp.int32) + j
            xj = plsc.load_gather(x_v, [xoff + colj])
            qj = plsc.load_gather(rows_v, [row16, colj])
            dj = xj - qj
            md16 = md16 + dj * dj
        idx16 = idx1[pl.ds(g * _L, _L)]
        md16 = jnp.where(idx16 < k, md16, 0.0)
        md_v[pl.ds(g * _L, _L)] = md16
        return sqacc + md16

    sqacc = lax.fori_loop(0, ngroups, _group, jnp.zeros((_L,), jnp.float32))
    sq_v[...] = sqacc
    pltpu.sync_copy(sq_v, sq_hbm.at[wid])
    pltpu.sync_copy(md_v, md_hbm.at[pl.ds(base, rpw)])

    plsc.subcore_barrier()
    for c in range(nchunks):
        pltpu.sync_copy(ones_v, hist_sh.at[idx_v.at[c]], add=True)
    plsc.subcore_barrier()

    @pl.when(sid == 0)
    def _emit():
        pltpu.sync_copy(hist_sh, hist_hbm.at[cid])


def _tc_final_body(hist_ref, sq_ref, sc1_ref, out_ref, *, k, d):
    counts = jnp.sum(hist_ref[...][:, :k], axis=0, keepdims=True)
    nv = sc1_ref[0]
    loss_vq = jnp.sum(sq_ref[...]) / (nv * d)
    p = counts / nv
    entropy = jnp.sum(p * jnp.log(p + 1e-10))
    perplexity = jnp.exp(-entropy)
    perplexity_loss = -jnp.log(perplexity + 1e-10)
    out_ref[0] = loss_vq + 0.01 * perplexity_loss
    out_ref[1] = loss_vq
    out_ref[2] = perplexity_loss
    out_ref[3] = perplexity


@jax.jit
def kernel(inputs, W):
    d = W.shape[1]
    K = W.shape[0]
    flat = inputs.reshape(-1, d)
    M = flat.shape[0]
    BM = 256
    nb = M // BM
    rpw = M // _NW
    hist_pad = K + 512

    idxm, idxg, sc1, w_aug = pl.pallas_call(
        functools.partial(_tc_dist_body, nb=nb, k=K, bm=BM),
        grid=(nb,),
        in_specs=[
            pl.BlockSpec((BM, d), lambda i: (i, 0)),
            pl.BlockSpec((K, d), lambda i: (0, 0)),
        ],
        out_specs=[
            pl.BlockSpec((M,), lambda i: (0,)),
            pl.BlockSpec((M,), lambda i: (0,)),
            pl.BlockSpec(memory_space=pltpu.SMEM),
            pl.BlockSpec((K + 8, d), lambda i: (0, 0)),
        ],
        out_shape=[
            jax.ShapeDtypeStruct((M,), jnp.int32),
            jax.ShapeDtypeStruct((M,), jnp.int32),
            jax.ShapeDtypeStruct((1,), jnp.float32),
            jax.ShapeDtypeStruct((K + 8, d), jnp.float32),
        ],
        scratch_shapes=[pltpu.VMEM((d, K), jnp.float32),
                        pltpu.VMEM((1, K), jnp.float32),
                        pltpu.SMEM((1,), jnp.float32)],
    )(flat, W)

    x1d = inputs.reshape(-1)

    qst, hist, md, sq = pl.kernel(
        functools.partial(_sc_gather_body, rpw=rpw, hist_pad=hist_pad,
                          k=K, d=d),
        out_type=[
            jax.ShapeDtypeStruct((M, d), jnp.float32),
            jax.ShapeDtypeStruct((_NC, hist_pad), jnp.float32),
            jax.ShapeDtypeStruct((M,), jnp.float32),
            jax.ShapeDtypeStruct((_NW, _L), jnp.float32),
        ],
        mesh=plsc.VectorSubcoreMesh(core_axis_name="c", subcore_axis_name="s"),
        scratch_types=[
            pltpu.VMEM((rpw // _CHUNK, _CHUNK), jnp.int32),
            pltpu.VMEM((rpw,), jnp.int32),
            pltpu.VMEM((rpw, d), jnp.float32),
            pltpu.VMEM((rpw * d,), jnp.float32),
            pltpu.VMEM((rpw,), jnp.float32),
            pltpu.VMEM((_L,), jnp.float32),
            pltpu.VMEM((_CHUNK,), jnp.float32),
            pltpu.VMEM((hist_pad // _NS,), jnp.float32),
            pltpu.VMEM_SHARED((hist_pad,), jnp.float32),
            pltpu.SemaphoreType.DMA,
        ],
        compiler_params=pltpu.CompilerParams(use_tc_tiling_on_sc=False,
                                             needs_layout_passes=False),
    )(idxg, w_aug, x1d)

    sc3 = pl.pallas_call(
        functools.partial(_tc_final_body, k=K, d=d),
        in_specs=[
            pl.BlockSpec((_NC, hist_pad), lambda: (0, 0)),
            pl.BlockSpec((_NW, _L), lambda: (0, 0)),
            pl.BlockSpec(memory_space=pltpu.SMEM),
        ],
        out_specs=pl.BlockSpec(memory_space=pltpu.SMEM),
        out_shape=jax.ShapeDtypeStruct((4,), jnp.float32),
    )(hist, sq, sc1)

    return (qst.reshape(inputs.shape), sc3[0], idxm,
            idxm.reshape(inputs.shape[:-1]), md.reshape(inputs.shape[:-1]),
            sc3[1], sc3[2], sc3[3])


# speedup vs baseline: 1.0376x; 1.0302x over previous
import functools

import jax
import jax.numpy as jnp
from jax import lax
from jax.experimental import pallas as pl
from jax.experimental.pallas import tpu as pltpu
from jax.experimental.pallas import tpu_sc as plsc

_NC, _NS = 2, 16
_NW = _NC * _NS
_CHUNK = 128
_L = 16


def _tc_dist_body(x_ref, w_ref, idxm_ref, idxg_ref, sc_ref, waug_ref,
                  wt2_ref, w2_ref, acc_ref, *, nb, k, bm):
    i = pl.program_id(0)

    @pl.when(i == 0)
    def _init():
        acc_ref[0] = 0.0
        wt2_ref[...] = jnp.transpose(w_ref[...]) * 2.0
        wt0 = wt2_ref[...]
        w2_ref[...] = jnp.sum(wt0 * wt0, axis=0, keepdims=True) * 0.25
        waug_ref[...] = jnp.concatenate(
            [w_ref[...], jnp.zeros((8, w_ref.shape[1]), jnp.float32)], axis=0)

    x = x_ref[...]
    x2 = jnp.sum(x * x, axis=1, keepdims=True)
    xw2 = lax.dot_general(x, wt2_ref[...], (((1,), (0,)), ((), ())),
                          preferred_element_type=jnp.float32)
    dist = (x2 + w2_ref[...]) - xw2

    idx = jnp.argmin(dist, axis=1, keepdims=True)
    valid = jnp.sqrt(x2) > 1e-06

    idx_t = jnp.reshape(idx, (bm,))
    valid_t = jnp.reshape(valid, (bm,))

    idxm_ref[pl.ds(i * bm, bm)] = jnp.where(valid_t, idx_t, 0)
    idxg_ref[pl.ds(i * bm, bm)] = jnp.where(valid_t, idx_t, k)
    acc_ref[0] += jnp.sum(valid_t.astype(jnp.float32))

    @pl.when(i == nb - 1)
    def _fin():
        sc_ref[0] = jnp.maximum(acc_ref[0], 1.0)


def _sc_gather_body(idxg_hbm, w_hbm, x_hbm, qst_hbm, hist_hbm, md_hbm, sq_hbm,
                    idx_v, idx1, rows_v, x_v, md_v, sq_v, ones_v, zeros_v,
                    hist_sh, sem, *, rpw, hist_pad, k, d):
    cid = lax.axis_index("c")
    sid = lax.axis_index("s")
    wid = sid * _NC + cid
    base = wid * rpw
    nchunks = rpw // _CHUNK
    ngroups = rpw // _L
    stripe = hist_pad // _NS

    zero16 = jnp.zeros((_L,), jnp.float32)

    def _zero(i, carry):
        zeros_v[pl.ds(i * _L, _L)] = zero16
        return carry

    lax.fori_loop(0, stripe // _L, _zero, 0)
    pltpu.sync_copy(zeros_v, hist_sh.at[pl.ds(sid * stripe, stripe)])

    one16 = jnp.ones((_L,), jnp.float32)

    def _ones(i, carry):
        ones_v[pl.ds(i * _L, _L)] = one16
        return carry

    lax.fori_loop(0, _CHUNK // _L, _ones, 0)

    for c in range(nchunks):
        pltpu.sync_copy(idxg_hbm.at[pl.ds(base + c * _CHUNK, _CHUNK)],
                        idx_v.at[c])
    pltpu.sync_copy(idxg_hbm.at[pl.ds(base, rpw)], idx1)
    pltpu.sync_copy(x_hbm.at[pl.ds(base * d, rpw * d)], x_v)
    for c in range(nchunks):
        pltpu.async_copy(w_hbm.at[idx_v.at[c]],
                         rows_v.at[pl.ds(c * _CHUNK, _CHUNK)], sem).wait()
    pltpu.sync_copy(rows_v, qst_hbm.at[pl.ds(base, rpw)])

    iota16 = lax.broadcasted_iota(jnp.int32, (_L,), 0)

    def _group(g, sqacc):
        row16 = iota16 + g * _L
        xoff = row16 * d

        md16 = jnp.zeros((_L,), jnp.float32)
        for j in range(d):
            colj = jnp.zeros((_L,), jnp.int32) + j
            xj = plsc.load_gather(x_v, [xoff + colj])
            qj = plsc.load_gather(rows_v, [row16, colj])
            dj = xj - qj
            md16 = md16 + dj * dj
        idx16 = idx1[pl.ds(g * _L, _L)]
        md16 = jnp.where(idx16 < k, md16, 0.0)
        md_v[pl.ds(g * _L, _L)] = md16
        return sqacc + md16

    sqacc = lax.fori_loop(0, ngroups, _group, jnp.zeros((_L,), jnp.float32))
    sq_v[...] = sqacc
    pltpu.sync_copy(sq_v, sq_hbm.at[wid])
    pltpu.sync_copy(md_v, md_hbm.at[pl.ds(base, rpw)])

    plsc.subcore_barrier()
    for c in range(nchunks):
        pltpu.sync_copy(ones_v, hist_sh.at[idx_v.at[c]], add=True)
    plsc.subcore_barrier()

    @pl.when(sid == 0)
    def _emit():
        pltpu.sync_copy(hist_sh, hist_hbm.at[cid])


def _tc_final_body(hist_ref, sq_ref, sc1_ref, out_ref, *, k, d):
    counts = jnp.sum(hist_ref[...][:, :k], axis=0, keepdims=True)
    nv = sc1_ref[0]
    loss_vq = jnp.sum(sq_ref[...]) / (nv * d)
    p = counts / nv
    entropy = jnp.sum(p * jnp.log(p + 1e-10))
    perplexity = jnp.exp(-entropy)
    perplexity_loss = -jnp.log(perplexity + 1e-10)
    out_ref[0] = loss_vq + 0.01 * perplexity_loss
    out_ref[1] = loss_vq
    out_ref[2] = perplexity_loss
    out_ref[3] = perplexity


@jax.jit
def kernel(inputs, W):
    d = W.shape[1]
    K = W.shape[0]
    flat = inputs.reshape(-1, d)
    M = flat.shape[0]
    BM = 512
    nb = M // BM
    rpw = M // _NW
    hist_pad = K + 512

    idxm, idxg, sc1, w_aug = pl.pallas_call(
        functools.partial(_tc_dist_body, nb=nb, k=K, bm=BM),
        grid=(nb,),
        in_specs=[
            pl.BlockSpec((BM, d), lambda i: (i, 0)),
            pl.BlockSpec((K, d), lambda i: (0, 0)),
        ],
        out_specs=[
            pl.BlockSpec((M,), lambda i: (0,)),
            pl.BlockSpec((M,), lambda i: (0,)),
            pl.BlockSpec(memory_space=pltpu.SMEM),
            pl.BlockSpec((K + 8, d), lambda i: (0, 0)),
        ],
        out_shape=[
            jax.ShapeDtypeStruct((M,), jnp.int32),
            jax.ShapeDtypeStruct((M,), jnp.int32),
            jax.ShapeDtypeStruct((1,), jnp.float32),
            jax.ShapeDtypeStruct((K + 8, d), jnp.float32),
        ],
        scratch_shapes=[pltpu.VMEM((d, K), jnp.float32),
                        pltpu.VMEM((1, K), jnp.float32),
                        pltpu.SMEM((1,), jnp.float32)],
    )(flat, W)

    x1d = inputs.reshape(-1)

    qst, hist, md, sq = pl.kernel(
        functools.partial(_sc_gather_body, rpw=rpw, hist_pad=hist_pad,
                          k=K, d=d),
        out_type=[
            jax.ShapeDtypeStruct((M, d), jnp.float32),
            jax.ShapeDtypeStruct((_NC, hist_pad), jnp.float32),
            jax.ShapeDtypeStruct((M,), jnp.float32),
            jax.ShapeDtypeStruct((_NW, _L), jnp.float32),
        ],
        mesh=plsc.VectorSubcoreMesh(core_axis_name="c", subcore_axis_name="s"),
        scratch_types=[
            pltpu.VMEM((rpw // _CHUNK, _CHUNK), jnp.int32),
            pltpu.VMEM((rpw,), jnp.int32),
            pltpu.VMEM((rpw, d), jnp.float32),
            pltpu.VMEM((rpw * d,), jnp.float32),
            pltpu.VMEM((rpw,), jnp.float32),
            pltpu.VMEM((_L,), jnp.float32),
            pltpu.VMEM((_CHUNK,), jnp.float32),
            pltpu.VMEM((hist_pad // _NS,), jnp.float32),
            pltpu.VMEM_SHARED((hist_pad,), jnp.float32),
            pltpu.SemaphoreType.DMA,
        ],
        compiler_params=pltpu.CompilerParams(use_tc_tiling_on_sc=False,
                                             needs_layout_passes=False),
    )(idxg, w_aug, x1d)

    sc3 = pl.pallas_call(
        functools.partial(_tc_final_body, k=K, d=d),
        in_specs=[
            pl.BlockSpec((_NC, hist_pad), lambda: (0, 0)),
            pl.BlockSpec((_NW, _L), lambda: (0, 0)),
            pl.BlockSpec(memory_space=pltpu.SMEM),
        ],
        out_specs=pl.BlockSpec(memory_space=pltpu.SMEM),
        out_shape=jax.ShapeDtypeStruct((4,), jnp.float32),
    )(hist, sq, sc1)

    return (qst.reshape(inputs.shape), sc3[0], idxm,
            idxm.reshape(inputs.shape[:-1]), md.reshape(inputs.shape[:-1]),
            sc3[1], sc3[2], sc3[3])


# speedup vs baseline: 1.0381x; 1.0005x over previous
import functools

import jax
import jax.numpy as jnp
from jax import lax
from jax.experimental import pallas as pl
from jax.experimental.pallas import tpu as pltpu
from jax.experimental.pallas import tpu_sc as plsc

_NC, _NS = 2, 16
_NW = _NC * _NS
_CHUNK = 128
_L = 16


def _tc_dist_body(x_ref, w_ref, idxm_ref, idxg_ref, sc_ref, waug_ref,
                  wt2_ref, w2_ref, acc_ref, *, nb, k, bm):
    i = pl.program_id(0)

    @pl.when(i == 0)
    def _init():
        acc_ref[0] = 0.0
        wt2_ref[...] = jnp.transpose(w_ref[...]) * 2.0
        wt0 = wt2_ref[...]
        w2_ref[...] = jnp.sum(wt0 * wt0, axis=0, keepdims=True) * 0.25
        waug_ref[...] = jnp.concatenate(
            [w_ref[...], jnp.zeros((8, w_ref.shape[1]), jnp.float32)], axis=0)

    x = x_ref[...]
    x2 = jnp.sum(x * x, axis=1, keepdims=True)
    xw2 = lax.dot_general(x, wt2_ref[...], (((1,), (0,)), ((), ())),
                          preferred_element_type=jnp.float32)
    dist = (x2 + w2_ref[...]) - xw2

    idx = jnp.argmin(dist, axis=1, keepdims=True)
    valid = jnp.sqrt(x2) > 1e-06

    idx_t = jnp.reshape(idx, (bm,))
    valid_t = jnp.reshape(valid, (bm,))

    idxm_ref[pl.ds(i * bm, bm)] = jnp.where(valid_t, idx_t, 0)
    idxg_ref[pl.ds(i * bm, bm)] = jnp.where(valid_t, idx_t, k)
    acc_ref[0] += jnp.sum(valid_t.astype(jnp.float32))

    @pl.when(i == nb - 1)
    def _fin():
        sc_ref[0] = jnp.maximum(acc_ref[0], 1.0)


def _sc_gather_body(idxg_hbm, w_hbm, x_hbm, qst_hbm, hist_hbm, md_hbm, sq_hbm,
                    idx_v, idx1, rows_v, x_v, md_v, sq_v, ones_v, zeros_v,
                    hist_sh, sem, *, rpw, hist_pad, k, d):
    cid = lax.axis_index("c")
    sid = lax.axis_index("s")
    wid = sid * _NC + cid
    base = wid * rpw
    nchunks = rpw // _CHUNK
    ngroups = rpw // _L
    stripe = hist_pad // _NS

    zero16 = jnp.zeros((_L,), jnp.float32)

    def _zero(i, carry):
        zeros_v[pl.ds(i * _L, _L)] = zero16
        return carry

    lax.fori_loop(0, stripe // _L, _zero, 0)
    pltpu.sync_copy(zeros_v, hist_sh.at[pl.ds(sid * stripe, stripe)])

    one16 = jnp.ones((_L,), jnp.float32)

    def _ones(i, carry):
        ones_v[pl.ds(i * _L, _L)] = one16
        return carry

    lax.fori_loop(0, _CHUNK // _L, _ones, 0)

    for c in range(nchunks):
        pltpu.sync_copy(idxg_hbm.at[pl.ds(base + c * _CHUNK, _CHUNK)],
                        idx_v.at[c])
    pltpu.sync_copy(idxg_hbm.at[pl.ds(base, rpw)], idx1)
    pltpu.sync_copy(x_hbm.at[pl.ds(base * d, rpw * d)], x_v)
    for c in range(nchunks):
        pltpu.async_copy(w_hbm.at[idx_v.at[c]],
                         rows_v.at[pl.ds(c * _CHUNK, _CHUNK)], sem).wait()
    pltpu.sync_copy(rows_v, qst_hbm.at[pl.ds(base, rpw)])

    iota16 = lax.broadcasted_iota(jnp.int32, (_L,), 0)

    def _group(g, sqacc):
        row16 = iota16 + g * _L
        xoff = row16 * d

        md16 = jnp.zeros((_L,), jnp.float32)
        for j in range(d):
            colj = jnp.zeros((_L,), jnp.int32) + j
            xj = plsc.load_gather(x_v, [xoff + colj])
            qj = plsc.load_gather(rows_v, [row16, colj])
            dj = xj - qj
            md16 = md16 + dj * dj
        idx16 = idx1[pl.ds(g * _L, _L)]
        md16 = jnp.where(idx16 < k, md16, 0.0)
        md_v[pl.ds(g * _L, _L)] = md16
        return sqacc + md16

    sqacc = lax.fori_loop(0, ngroups, _group, jnp.zeros((_L,), jnp.float32))
    sq_v[...] = sqacc
    pltpu.sync_copy(sq_v, sq_hbm.at[wid])
    pltpu.sync_copy(md_v, md_hbm.at[pl.ds(base, rpw)])

    plsc.subcore_barrier()
    for c in range(nchunks):
        pltpu.sync_copy(ones_v, hist_sh.at[idx_v.at[c]], add=True)
    plsc.subcore_barrier()

    @pl.when(sid == 0)
    def _emit():
        pltpu.sync_copy(hist_sh, hist_hbm.at[cid])


def _tc_final_body(hist_ref, sq_ref, sc1_ref, out_ref, *, k, d):
    counts = jnp.sum(hist_ref[...][:, :k], axis=0, keepdims=True)
    nv = sc1_ref[0]
    loss_vq = jnp.sum(sq_ref[...]) / (nv * d)
    p = counts / nv
    entropy = jnp.sum(p * jnp.log(p + 1e-10))
    perplexity = jnp.exp(-entropy)
    perplexity_loss = -jnp.log(perplexity + 1e-10)
    out_ref[0] = loss_vq + 0.01 * perplexity_loss
    out_ref[1] = loss_vq
    out_ref[2] = perplexity_loss
    out_ref[3] = perplexity


@jax.jit
def kernel(inputs, W):
    d = W.shape[1]
    K = W.shape[0]
    flat = inputs.reshape(-1, d)
    M = flat.shape[0]
    BM = 1024
    nb = M // BM
    rpw = M // _NW
    hist_pad = K + 512

    idxm, idxg, sc1, w_aug = pl.pallas_call(
        functools.partial(_tc_dist_body, nb=nb, k=K, bm=BM),
        grid=(nb,),
        in_specs=[
            pl.BlockSpec((BM, d), lambda i: (i, 0)),
            pl.BlockSpec((K, d), lambda i: (0, 0)),
        ],
        out_specs=[
            pl.BlockSpec((M,), lambda i: (0,)),
            pl.BlockSpec((M,), lambda i: (0,)),
            pl.BlockSpec(memory_space=pltpu.SMEM),
            pl.BlockSpec((K + 8, d), lambda i: (0, 0)),
        ],
        out_shape=[
            jax.ShapeDtypeStruct((M,), jnp.int32),
            jax.ShapeDtypeStruct((M,), jnp.int32),
            jax.ShapeDtypeStruct((1,), jnp.float32),
            jax.ShapeDtypeStruct((K + 8, d), jnp.float32),
        ],
        scratch_shapes=[pltpu.VMEM((d, K), jnp.float32),
                        pltpu.VMEM((1, K), jnp.float32),
                        pltpu.SMEM((1,), jnp.float32)],
        compiler_params=pltpu.CompilerParams(
            vmem_limit_bytes=112 * 1024 * 1024),
    )(flat, W)

    x1d = inputs.reshape(-1)

    qst, hist, md, sq = pl.kernel(
        functools.partial(_sc_gather_body, rpw=rpw, hist_pad=hist_pad,
                          k=K, d=d),
        out_type=[
            jax.ShapeDtypeStruct((M, d), jnp.float32),
            jax.ShapeDtypeStruct((_NC, hist_pad), jnp.float32),
            jax.ShapeDtypeStruct((M,), jnp.float32),
            jax.ShapeDtypeStruct((_NW, _L), jnp.float32),
        ],
        mesh=plsc.VectorSubcoreMesh(core_axis_name="c", subcore_axis_name="s"),
        scratch_types=[
            pltpu.VMEM((rpw // _CHUNK, _CHUNK), jnp.int32),
            pltpu.VMEM((rpw,), jnp.int32),
            pltpu.VMEM((rpw, d), jnp.float32),
            pltpu.VMEM((rpw * d,), jnp.float32),
            pltpu.VMEM((rpw,), jnp.float32),
            pltpu.VMEM((_L,), jnp.float32),
            pltpu.VMEM((_CHUNK,), jnp.float32),
            pltpu.VMEM((hist_pad // _NS,), jnp.float32),
            pltpu.VMEM_SHARED((hist_pad,), jnp.float32),
            pltpu.SemaphoreType.DMA,
        ],
        compiler_params=pltpu.CompilerParams(use_tc_tiling_on_sc=False,
                                             needs_layout_passes=False),
    )(idxg, w_aug, x1d)

    sc3 = pl.pallas_call(
        functools.partial(_tc_final_body, k=K, d=d),
        in_specs=[
            pl.BlockSpec((_NC, hist_pad), lambda: (0, 0)),
            pl.BlockSpec((_NW, _L), lambda: (0, 0)),
            pl.BlockSpec(memory_space=pltpu.SMEM),
        ],
        out_specs=pl.BlockSpec(memory_space=pltpu.SMEM),
        out_shape=jax.ShapeDtypeStruct((4,), jnp.float32),
    )(hist, sq, sc1)

    return (qst.reshape(inputs.shape), sc3[0], idxm,
            idxm.reshape(inputs.shape[:-1]), md.reshape(inputs.shape[:-1]),
            sc3[1], sc3[2], sc3[3])
